# baseline (speedup 1.0000x reference)
import functools

import jax
import jax.numpy as jnp
from jax import lax
from jax.experimental import pallas as pl
from jax.experimental.pallas import tpu as pltpu
from jax.experimental.pallas import tpu_sc as plsc

_EMBED = 64
_HIDDEN = 128
_BATCH = 16384

_NC = 2
_NS = 16
_NW = _NC * _NS
_BPW = _BATCH // _NW
_CH = 128
_NCH = _BPW // _CH


def _sc_gather(table, idx):
  mesh = plsc.VectorSubcoreMesh(core_axis_name="c", subcore_axis_name="s")

  @functools.partial(
      pl.kernel,
      mesh=mesh,
      out_type=jax.ShapeDtypeStruct((_BATCH, _EMBED), jnp.float32),
      compiler_params=pltpu.CompilerParams(use_tc_tiling_on_sc=False),
      scratch_types=[
          pltpu.VMEM((_NCH, _CH), jnp.int32),
          pltpu.VMEM((_BPW, _EMBED), jnp.float32),
          pltpu.SemaphoreType.DMA,
      ],
  )
  def k(table_hbm, idx_hbm, out_hbm, idx_v, rows_v, sem):
    wid = lax.axis_index("s") * _NC + lax.axis_index("c")
    base = wid * _BPW
    pltpu.sync_copy(idx_hbm.at[wid], idx_v)
    cps = [
        pltpu.async_copy(
            table_hbm.at[idx_v.at[j]],
            rows_v.at[pl.ds(j * _CH, _CH)],
            sem,
        )
        for j in range(_NCH)
    ]
    for cp in cps:
      cp.wait()
    pltpu.sync_copy(rows_v, out_hbm.at[pl.ds(base, _BPW)])

  return k(table, idx)


def _mlp_body(x_ref, w1_ref, b1_ref, w2_ref, b2_ref, out_ref):
  x = x_ref[...]
  h = jnp.dot(x, w1_ref[...], preferred_element_type=jnp.float32)
  h = jnp.maximum(h + b1_ref[...], 0.0)
  out = jnp.dot(h, w2_ref[...], preferred_element_type=jnp.float32)
  out_ref[...] = out + b2_ref[...]


def _tc_mlp(x, W1, b1, W2, b2):
  bb = 2048
  return pl.pallas_call(
      _mlp_body,
      grid=(_BATCH // bb,),
      in_specs=[
          pl.BlockSpec((bb, _EMBED), lambda i: (i, 0)),
          pl.BlockSpec((_EMBED, _HIDDEN), lambda i: (0, 0)),
          pl.BlockSpec((1, _HIDDEN), lambda i: (0, 0)),
          pl.BlockSpec((_HIDDEN, _EMBED), lambda i: (0, 0)),
          pl.BlockSpec((1, _EMBED), lambda i: (0, 0)),
      ],
      out_specs=pl.BlockSpec((bb, _EMBED), lambda i: (i, 0)),
      out_shape=jax.ShapeDtypeStruct((_BATCH, _EMBED), jnp.float32),
  )(x, W1, b1.reshape(1, _HIDDEN), W2, b2.reshape(1, _EMBED))


def kernel(nodes, table, W1, b1, W2, b2):
  idx = nodes.astype(jnp.int32).reshape(_NW, _NCH, _CH)
  x = _sc_gather(table, idx)
  return _tc_mlp(x, W1, b1, W2, b2)


# speedup vs baseline: 1.6898x; 1.6898x over previous
import functools

import jax
import jax.numpy as jnp
from jax import lax
from jax.experimental import pallas as pl
from jax.experimental.pallas import tpu as pltpu
from jax.experimental.pallas import tpu_sc as plsc

_EMBED = 64
_HIDDEN = 128
_BATCH = 16384

_NC = 2
_NS = 16
_NW = _NC * _NS
_BPW = _BATCH // _NW
_WIN = 32

_ROW_BYTES = _EMBED * 4


def _sc_gather(table, idx):
  mesh = plsc.VectorSubcoreMesh(core_axis_name="c", subcore_axis_name="s")

  @functools.partial(
      pl.kernel,
      mesh=mesh,
      out_type=jax.ShapeDtypeStruct((_BATCH, _EMBED), jnp.float32),
      scratch_types=[
          pltpu.VMEM((_BPW,), jnp.int32),
          pltpu.VMEM((_BPW, _EMBED), jnp.float32),
          pltpu.SemaphoreType.DMA,
      ],
  )
  def k(table_hbm, idx_hbm, out_hbm, idx_v, rows_v, sem):
    wid = lax.axis_index("s") * _NC + lax.axis_index("c")
    pltpu.sync_copy(idx_hbm.at[wid], idx_v)

    def body(g, carry):
      v = idx_v[pl.ds(g * 16, 16)]
      base = g * 16
      for l in range(16):
        t = v[l]
        pltpu.make_async_copy(table_hbm.at[t], rows_v.at[base + l], sem).start()

      @pl.when(g >= 2)
      def _():
        pltpu.make_async_copy(
            table_hbm.at[pl.ds(0, 16)],
            rows_v.at[pl.ds((g - 2) * 16, 16)],
            sem,
        ).wait()

      return carry

    ngroups = _BPW // 16
    lax.fori_loop(0, ngroups, body, 0)
    pltpu.make_async_copy(
        table_hbm.at[pl.ds(0, 32)],
        rows_v.at[pl.ds(_BPW - 32, 32)],
        sem,
    ).wait()
    pltpu.sync_copy(rows_v, out_hbm.at[pl.ds(wid * _BPW, _BPW)])

  return k(table, idx)


def _mlp_body(x_ref, w1_ref, b1_ref, w2_ref, b2_ref, out_ref):
  x = x_ref[...]
  h = jnp.dot(x, w1_ref[...], preferred_element_type=jnp.float32)
  h = jnp.maximum(h + b1_ref[...], 0.0)
  out = jnp.dot(h, w2_ref[...], preferred_element_type=jnp.float32)
  out_ref[...] = out + b2_ref[...]


def _tc_mlp(x, W1, b1, W2, b2):
  bb = 2048
  return pl.pallas_call(
      _mlp_body,
      grid=(_BATCH // bb,),
      in_specs=[
          pl.BlockSpec((bb, _EMBED), lambda i: (i, 0)),
          pl.BlockSpec((_EMBED, _HIDDEN), lambda i: (0, 0)),
          pl.BlockSpec((1, _HIDDEN), lambda i: (0, 0)),
          pl.BlockSpec((_HIDDEN, _EMBED), lambda i: (0, 0)),
          pl.BlockSpec((1, _EMBED), lambda i: (0, 0)),
      ],
      out_specs=pl.BlockSpec((bb, _EMBED), lambda i: (i, 0)),
      out_shape=jax.ShapeDtypeStruct((_BATCH, _EMBED), jnp.float32),
  )(x, W1, b1.reshape(1, _HIDDEN), W2, b2.reshape(1, _EMBED))


def kernel(nodes, table, W1, b1, W2, b2):
  idx = nodes.astype(jnp.int32).reshape(_NW, _BPW)
  x = _sc_gather(table, idx)
  return _tc_mlp(x, W1, b1, W2, b2)
